# baseline (speedup 1.0000x reference)
import functools

import jax
import jax.numpy as jnp
from jax import lax
from jax.experimental import pallas as pl
from jax.experimental.pallas import tpu as pltpu
from jax.experimental.pallas import tpu_sc as plsc

K = 1024
D = 256
N = 9216
BLK = 1024
NB = N // BLK


def _vq_block_kernel(z_ref, esq_ref, emb_ref, ar_ref, idx_ref, dmin_ref):
    zb = z_ref[...]
    zsq = jnp.sum(zb * zb, axis=1)
    m = jax.lax.dot_general(
        zb, emb_ref[...], (((1,), (1,)), ((), ())),
        preferred_element_type=jnp.float32)
    d = (zsq[:, None] + esq_ref[...][None, :]) - 2.0 * m
    dmin = jnp.min(d, axis=1)
    arange = ar_ref[...]
    sel = jnp.where(d == dmin[:, None], arange, jnp.float32(K))
    idx_f = jnp.min(sel, axis=1)
    idx_ref[...] = idx_f.astype(jnp.int32)
    dmin_ref[...] = dmin


def _make_sc_gather():
    info = plsc.get_sparse_core_info()
    nw = info.num_cores * info.num_subcores
    b_per_w = N // nw
    mesh = plsc.VectorSubcoreMesh(core_axis_name="c", subcore_axis_name="s")

    @functools.partial(
        pl.kernel, mesh=mesh,
        out_type=jax.ShapeDtypeStruct((N, D), jnp.float32),
        scratch_types=[
            pltpu.VMEM((b_per_w,), jnp.int32),
            pltpu.VMEM((b_per_w, D), jnp.float32),
            pltpu.SemaphoreType.DMA,
        ],
    )
    def sc_gather(table_hbm, idx_hbm, out_hbm, idx_v, rows_v, sem):
        wid = lax.axis_index("s") * info.num_cores + lax.axis_index("c")
        base = wid * b_per_w
        pltpu.sync_copy(idx_hbm.at[pl.ds(base, b_per_w)], idx_v)
        pltpu.async_copy(table_hbm.at[idx_v], rows_v, sem).wait()
        pltpu.sync_copy(rows_v, out_hbm.at[pl.ds(base, b_per_w)])

    return sc_gather


def kernel(z, emb):
    B, Dd, H, W = z.shape
    z_flat = jnp.transpose(z, (0, 2, 3, 1)).reshape(-1, Dd)
    esq = jnp.sum(emb ** 2, axis=1)

    idx, dmin = pl.pallas_call(
        _vq_block_kernel,
        grid=(NB,),
        in_specs=[
            pl.BlockSpec((BLK, D), lambda i: (i, 0)),
            pl.BlockSpec((K,), lambda i: (0,)),
            pl.BlockSpec((K, D), lambda i: (0, 0)),
            pl.BlockSpec((1, K), lambda i: (0, 0)),
        ],
        out_specs=[
            pl.BlockSpec((BLK,), lambda i: (i,)),
            pl.BlockSpec((BLK,), lambda i: (i,)),
        ],
        out_shape=[
            jax.ShapeDtypeStruct((N,), jnp.int32),
            jax.ShapeDtypeStruct((N,), jnp.float32),
        ],
    )(z_flat, esq, emb, jnp.arange(K, dtype=jnp.float32)[None, :])

    zq_flat = _make_sc_gather()(emb, idx)

    commitment_loss = 0.25 * (jnp.sum(dmin) / (N * D))
    z_q_out = jnp.transpose(zq_flat.reshape(B, H, W, Dd), (0, 3, 1, 2))
    indices_out = idx.reshape(B, H, W)
    return (z_q_out, commitment_loss, indices_out)


# speedup vs baseline: 1.2321x; 1.2321x over previous
import jax
import jax.numpy as jnp
import numpy as np
from jax.experimental import pallas as pl

K = 1024
D = 256
N = 9216
BLK = 512
NB = N // BLK


def _vq_block_kernel(z_ref, esq_ref, ehi_ref, elo_ref, emb_ref, ar_ref,
                     zq_ref, idx_ref, dmin_ref):
    zb = z_ref[...]
    zsq = jnp.sum(zb * zb, axis=1)
    m = jax.lax.dot_general(
        zb, emb_ref[...], (((1,), (1,)), ((), ())),
        preferred_element_type=jnp.float32)
    d = (zsq[:, None] + esq_ref[...][None, :]) - 2.0 * m
    dmin = jnp.min(d, axis=1)
    arange = ar_ref[...]
    sel = jnp.where(d == dmin[:, None], arange, jnp.float32(K))
    idx_f = jnp.min(sel, axis=1)
    oh = (arange == idx_f[:, None]).astype(jnp.float32)
    zq = (jax.lax.dot_general(oh, ehi_ref[...], (((1,), (0,)), ((), ())),
                              preferred_element_type=jnp.float32)
          + jax.lax.dot_general(oh, elo_ref[...], (((1,), (0,)), ((), ())),
                                preferred_element_type=jnp.float32))
    zq_ref[...] = zq
    idx_ref[...] = idx_f.astype(jnp.int32)
    dmin_ref[...] = dmin


def kernel(z, emb):
    B, Dd, H, W = z.shape
    z_flat = jnp.transpose(z, (0, 2, 3, 1)).reshape(-1, Dd)
    esq = jnp.sum(emb ** 2, axis=1)
    e_hi = jax.lax.bitcast_convert_type(
        jax.lax.bitcast_convert_type(emb, jnp.uint32) & jnp.uint32(0xFFFF0000),
        jnp.float32)
    e_lo = emb - e_hi

    zq_flat, idx, dmin = pl.pallas_call(
        _vq_block_kernel,
        grid=(NB,),
        in_specs=[
            pl.BlockSpec((BLK, D), lambda i: (i, 0)),
            pl.BlockSpec((K,), lambda i: (0,)),
            pl.BlockSpec((K, D), lambda i: (0, 0)),
            pl.BlockSpec((K, D), lambda i: (0, 0)),
            pl.BlockSpec((K, D), lambda i: (0, 0)),
            pl.BlockSpec((1, K), lambda i: (0, 0)),
        ],
        out_specs=[
            pl.BlockSpec((BLK, D), lambda i: (i, 0)),
            pl.BlockSpec((BLK,), lambda i: (i,)),
            pl.BlockSpec((BLK,), lambda i: (i,)),
        ],
        out_shape=[
            jax.ShapeDtypeStruct((N, D), jnp.float32),
            jax.ShapeDtypeStruct((N,), jnp.int32),
            jax.ShapeDtypeStruct((N,), jnp.float32),
        ],
    )(z_flat, esq, e_hi, e_lo, emb,
      jnp.arange(K, dtype=jnp.float32)[None, :])

    commitment_loss = 0.25 * (jnp.sum(dmin) / (N * D))
    z_q_out = jnp.transpose(zq_flat.reshape(B, H, W, Dd), (0, 3, 1, 2))
    indices_out = idx.reshape(B, H, W)
    return (z_q_out, commitment_loss, indices_out)


# speedup vs baseline: 1.3940x; 1.1314x over previous
import jax
import jax.numpy as jnp
import numpy as np
from jax.experimental import pallas as pl

K = 1024
D = 256
N = 9216
BLK = 3072
NB = N // BLK


def _vq_block_kernel(z_ref, esq_ref, ehi_ref, elo_ref, emb_ref, ar_ref,
                     zq_ref, idx_ref, dmin_ref):
    zb = z_ref[...]
    zsq = jnp.sum(zb * zb, axis=1)
    m = jax.lax.dot_general(
        zb, emb_ref[...], (((1,), (1,)), ((), ())),
        preferred_element_type=jnp.float32)
    d = (zsq[:, None] + esq_ref[...][None, :]) - 2.0 * m
    dmin = jnp.min(d, axis=1)
    arange = ar_ref[...]
    sel = jnp.where(d == dmin[:, None], arange, jnp.float32(K))
    idx_f = jnp.min(sel, axis=1)
    oh = (arange == idx_f[:, None]).astype(jnp.float32)
    zq = (jax.lax.dot_general(oh, ehi_ref[...], (((1,), (0,)), ((), ())),
                              preferred_element_type=jnp.float32)
          + jax.lax.dot_general(oh, elo_ref[...], (((1,), (0,)), ((), ())),
                                preferred_element_type=jnp.float32))
    zq_ref[...] = zq
    idx_ref[...] = idx_f.astype(jnp.int32)
    dmin_ref[...] = dmin


def kernel(z, emb):
    B, Dd, H, W = z.shape
    z_flat = jnp.transpose(z, (0, 2, 3, 1)).reshape(-1, Dd)
    esq = jnp.sum(emb ** 2, axis=1)
    e_hi = jax.lax.bitcast_convert_type(
        jax.lax.bitcast_convert_type(emb, jnp.uint32) & jnp.uint32(0xFFFF0000),
        jnp.float32)
    e_lo = emb - e_hi

    zq_flat, idx, dmin = pl.pallas_call(
        _vq_block_kernel,
        grid=(NB,),
        in_specs=[
            pl.BlockSpec((BLK, D), lambda i: (i, 0)),
            pl.BlockSpec((K,), lambda i: (0,)),
            pl.BlockSpec((K, D), lambda i: (0, 0)),
            pl.BlockSpec((K, D), lambda i: (0, 0)),
            pl.BlockSpec((K, D), lambda i: (0, 0)),
            pl.BlockSpec((1, K), lambda i: (0, 0)),
        ],
        out_specs=[
            pl.BlockSpec((BLK, D), lambda i: (i, 0)),
            pl.BlockSpec((BLK,), lambda i: (i,)),
            pl.BlockSpec((BLK,), lambda i: (i,)),
        ],
        out_shape=[
            jax.ShapeDtypeStruct((N, D), jnp.float32),
            jax.ShapeDtypeStruct((N,), jnp.int32),
            jax.ShapeDtypeStruct((N,), jnp.float32),
        ],
    )(z_flat, esq, e_hi, e_lo, emb,
      jnp.arange(K, dtype=jnp.float32)[None, :])

    commitment_loss = 0.25 * (jnp.sum(dmin) / (N * D))
    z_q_out = jnp.transpose(zq_flat.reshape(B, H, W, Dd), (0, 3, 1, 2))
    indices_out = idx.reshape(B, H, W)
    return (z_q_out, commitment_loss, indices_out)


# speedup vs baseline: 1.5803x; 1.1337x over previous
import jax
import jax.numpy as jnp
import numpy as np
from jax.experimental import pallas as pl

K = 1024
D = 256
N = 9216
BLK = 3072
NB = N // BLK


def _vq_block_kernel(z_ref, esq_ref, ehi_ref, elo_ref, emb_ref, ar_ref,
                     zq_ref, idx_ref, dmin_ref):
    zb = z_ref[...]
    zsq = jnp.sum(zb * zb, axis=1)
    m = jax.lax.dot_general(
        zb, emb_ref[...], (((1,), (1,)), ((), ())),
        preferred_element_type=jnp.float32)
    d = (zsq[:, None] + esq_ref[...][None, :]) - 2.0 * m
    dmin = jnp.min(d, axis=1)
    arange = ar_ref[...]
    sel = jnp.where(d == dmin[:, None], arange, jnp.float32(K))
    idx_f = jnp.min(sel, axis=1)
    oh = (arange == idx_f[:, None]).astype(jnp.float32)
    zq = jax.lax.dot_general(oh, ehi_ref[...], (((1,), (0,)), ((), ())),
                             preferred_element_type=jnp.float32)
    del elo_ref
    zq_ref[...] = zq
    idx_ref[...] = idx_f.astype(jnp.int32)
    dmin_ref[...] = dmin


def kernel(z, emb):
    B, Dd, H, W = z.shape
    z_flat = jnp.transpose(z, (0, 2, 3, 1)).reshape(-1, Dd)
    esq = jnp.sum(emb ** 2, axis=1)
    e_hi = jax.lax.bitcast_convert_type(
        jax.lax.bitcast_convert_type(emb, jnp.uint32) & jnp.uint32(0xFFFF0000),
        jnp.float32)
    e_lo = emb - e_hi

    zq_flat, idx, dmin = pl.pallas_call(
        _vq_block_kernel,
        grid=(NB,),
        in_specs=[
            pl.BlockSpec((BLK, D), lambda i: (i, 0)),
            pl.BlockSpec((K,), lambda i: (0,)),
            pl.BlockSpec((K, D), lambda i: (0, 0)),
            pl.BlockSpec((K, D), lambda i: (0, 0)),
            pl.BlockSpec((K, D), lambda i: (0, 0)),
            pl.BlockSpec((1, K), lambda i: (0, 0)),
        ],
        out_specs=[
            pl.BlockSpec((BLK, D), lambda i: (i, 0)),
            pl.BlockSpec((BLK,), lambda i: (i,)),
            pl.BlockSpec((BLK,), lambda i: (i,)),
        ],
        out_shape=[
            jax.ShapeDtypeStruct((N, D), jnp.float32),
            jax.ShapeDtypeStruct((N,), jnp.int32),
            jax.ShapeDtypeStruct((N,), jnp.float32),
        ],
    )(z_flat, esq, e_hi, e_lo, emb,
      jnp.arange(K, dtype=jnp.float32)[None, :])

    commitment_loss = 0.25 * (jnp.sum(dmin) / (N * D))
    z_q_out = jnp.transpose(zq_flat.reshape(B, H, W, Dd), (0, 3, 1, 2))
    indices_out = idx.reshape(B, H, W)
    return (z_q_out, commitment_loss, indices_out)


# speedup vs baseline: 1.6218x; 1.0263x over previous
import jax
import jax.numpy as jnp
from jax.experimental import pallas as pl

K = 1024
D = 256
N = 9216
BLK = 3072
NB = N // BLK


def _vq_block_kernel(z_ref, esq_ref, ebf_ref, emb_ref, ar_ref,
                     zq_ref, idx_ref, dmin_ref):
    zb = z_ref[...]
    zsq = jnp.sum(zb * zb, axis=1)
    m = jax.lax.dot_general(
        zb, emb_ref[...], (((1,), (1,)), ((), ())),
        preferred_element_type=jnp.float32)
    d = (zsq[:, None] + esq_ref[...][None, :]) - 2.0 * m
    dmin = jnp.min(d, axis=1)
    arange = ar_ref[...]
    sel = jnp.where(d == dmin[:, None], arange, jnp.float32(K))
    idx_f = jnp.min(sel, axis=1)
    oh = (arange == idx_f[:, None]).astype(jnp.bfloat16)
    zq = jax.lax.dot_general(oh, ebf_ref[...], (((1,), (0,)), ((), ())),
                             preferred_element_type=jnp.float32)
    zq_ref[...] = zq
    idx_ref[...] = idx_f.astype(jnp.int32)
    dmin_ref[...] = dmin


def kernel(z, emb):
    B, Dd, H, W = z.shape
    z_flat = jnp.transpose(z, (0, 2, 3, 1)).reshape(-1, Dd)
    esq = jnp.sum(emb ** 2, axis=1)
    e_bf = emb.astype(jnp.bfloat16)

    zq_flat, idx, dmin = pl.pallas_call(
        _vq_block_kernel,
        grid=(NB,),
        in_specs=[
            pl.BlockSpec((BLK, D), lambda i: (i, 0)),
            pl.BlockSpec((K,), lambda i: (0,)),
            pl.BlockSpec((K, D), lambda i: (0, 0)),
            pl.BlockSpec((K, D), lambda i: (0, 0)),
            pl.BlockSpec((1, K), lambda i: (0, 0)),
        ],
        out_specs=[
            pl.BlockSpec((BLK, D), lambda i: (i, 0)),
            pl.BlockSpec((BLK,), lambda i: (i,)),
            pl.BlockSpec((BLK,), lambda i: (i,)),
        ],
        out_shape=[
            jax.ShapeDtypeStruct((N, D), jnp.float32),
            jax.ShapeDtypeStruct((N,), jnp.int32),
            jax.ShapeDtypeStruct((N,), jnp.float32),
        ],
    )(z_flat, esq, e_bf, emb,
      jnp.arange(K, dtype=jnp.float32)[None, :])

    commitment_loss = 0.25 * (jnp.sum(dmin) / (N * D))
    z_q_out = jnp.transpose(zq_flat.reshape(B, H, W, Dd), (0, 3, 1, 2))
    indices_out = idx.reshape(B, H, W)
    return (z_q_out, commitment_loss, indices_out)
